# baseline (speedup 1.0000x reference)
import functools

import jax
import jax.numpy as jnp
from jax import lax
from jax.experimental import pallas as pl
from jax.experimental.pallas import tpu as pltpu
from jax.experimental.pallas import tpu_sc as plsc

N = 10000
E = 160000
D = 256
K = 30
KP = 32
TW = 128
BN = 1000
NBLK = N // BN
W = 640
E2 = 163840
NWIN = E2 // W
WSUB = NWIN // 16
NEG = -1e30


def _s_body(x_ref, wp_ref, b_ref, s_ref):
    logits = jnp.dot(x_ref[...], wp_ref[...],
                     preferred_element_type=jnp.float32) + b_ref[0:1, :]
    m = jnp.max(logits, axis=-1, keepdims=True)
    e = jnp.exp(logits - m)
    s_ref[...] = e / jnp.sum(e, axis=-1, keepdims=True)


def _stats_body(s_ref, x_ref, c_ref, ss_ref):
    i = pl.program_id(0)
    s = s_ref[...][:, :KP]
    ci = lax.broadcasted_iota(jnp.int32, (BN, KP), 1)
    s_aug = jnp.where(ci == KP - 1, 1.0, s)
    c_part = lax.dot_general(s, x_ref[...], (((0,), (0,)), ((), ())),
                             preferred_element_type=jnp.float32)
    ss_part = lax.dot_general(s, s_aug, (((0,), (0,)), ((), ())),
                              preferred_element_type=jnp.float32)

    @pl.when(i == 0)
    def _():
        c_ref[...] = c_part
        ss_ref[...] = ss_part

    @pl.when(i != 0)
    def _():
        c_ref[...] += c_part
        ss_ref[...] += ss_part


def _final_body(s_ref, a0_ref, ss_ref, c_ref, pw_ref, pb_ref,
                adj_ref, z_ref, scal_ref):
    i = pl.program_id(0)
    s = s_ref[...][:, :KP]
    adj_s = a0_ref[0][:, :KP]
    ci = lax.broadcasted_iota(jnp.int32, (BN, KP), 1)
    rs_s = jnp.sum(s, axis=1, keepdims=True)
    rs_a = jnp.sum(adj_s, axis=1, keepdims=True)
    s_aug = jnp.where(ci == KP - 1, rs_s, s)
    a_aug = jnp.where(ci == KP - 1, rs_a, adj_s)
    part = lax.dot_general(s_aug, a_aug, (((0,), (0,)), ((), ())),
                           preferred_element_type=jnp.float32)

    @pl.when(i == 0)
    def _():
        adj_ref[...] = part

    @pl.when(i != 0)
    def _():
        adj_ref[...] += part

    @pl.when(i == NBLK - 1)
    def _():
        r32 = lax.broadcasted_iota(jnp.int32, (KP, KP), 0)
        c32 = lax.broadcasted_iota(jnp.int32, (KP, KP), 1)
        adjn = adj_ref[...]
        cut = jnp.sum(jnp.where((r32 == c32) & (r32 < K), adjn, 0.0))
        vol = jnp.sum(jnp.where((r32 == KP - 1) & (c32 == KP - 1), adjn, 0.0))
        mincut = -cut / (vol + 1e-9)
        ss = ss_ref[...]
        eye = jnp.where((r32 == c32) & (r32 < K), 1.0, 0.0)
        diff = jnp.where((r32 < K) & (c32 < K), ss - eye, 0.0)
        ortho = jnp.sqrt(jnp.sum(diff * diff))
        colsum = ss[:, KP - 1:KP]
        z = jnp.dot(c_ref[...], pw_ref[...],
                    preferred_element_type=jnp.float32)
        z_ref[...] = z + colsum * pb_ref[0:1, :]
        r8 = lax.broadcasted_iota(jnp.int32, (8, 128), 0)
        c8 = lax.broadcasted_iota(jnp.int32, (8, 128), 1)
        scal_ref[...] = jnp.where((r8 == 0) & (c8 == 0), mincut,
                                  jnp.where((r8 == 0) & (c8 == 1), ortho, 0.0))


def _compute_s(x, wa_pad, ba8):
    return pl.pallas_call(
        _s_body,
        grid=(NBLK,),
        in_specs=[
            pl.BlockSpec((BN, D), lambda i: (i, 0)),
            pl.BlockSpec((D, TW), lambda i: (0, 0)),
            pl.BlockSpec((8, TW), lambda i: (0, 0)),
        ],
        out_specs=pl.BlockSpec((BN, TW), lambda i: (i, 0)),
        out_shape=jax.ShapeDtypeStruct((N, TW), jnp.float32),
    )(x, wa_pad, ba8)


def _compute_stats(s128, x):
    return pl.pallas_call(
        _stats_body,
        grid=(NBLK,),
        in_specs=[
            pl.BlockSpec((BN, TW), lambda i: (i, 0)),
            pl.BlockSpec((BN, D), lambda i: (i, 0)),
        ],
        out_specs=[
            pl.BlockSpec((KP, D), lambda i: (0, 0)),
            pl.BlockSpec((KP, KP), lambda i: (0, 0)),
        ],
        out_shape=[
            jax.ShapeDtypeStruct((KP, D), jnp.float32),
            jax.ShapeDtypeStruct((KP, KP), jnp.float32),
        ],
    )(s128, x)


NH = N // 2
ACC = NH + 8


def _sc_segsum(s128, scr, scc, zeros):
    mesh = plsc.VectorSubcoreMesh(core_axis_name="c", subcore_axis_name="s")

    @functools.partial(
        pl.kernel,
        out_type=jax.ShapeDtypeStruct((2, NH, TW), jnp.float32),
        mesh=mesh,
        scratch_types=[
            pltpu.VMEM((W,), jnp.int32),
            pltpu.VMEM((W,), jnp.int32),
            pltpu.VMEM((W,), jnp.int32),
            pltpu.VMEM((W,), jnp.int32),
            pltpu.VMEM((W, TW), jnp.float32),
            pltpu.VMEM_SHARED((ACC, TW), jnp.float32),
            pltpu.SemaphoreType.DMA,
            pltpu.SemaphoreType.DMA,
            pltpu.SemaphoreType.DMA,
            pltpu.SemaphoreType.DMA,
        ],
    )
    def k(t_hbm, r_hbm, c_hbm, z_hbm, out_hbm,
          idxr0, idxr1, idxc0, idxc1, buf, acc, ir0, ir1, ic0, ic1):
        idxr = (idxr0, idxr1)
        idxc = (idxc0, idxc1)
        ir = (ir0, ir1)
        ic = (ic0, ic1)
        c = lax.axis_index("c")
        s = lax.axis_index("s")

        def rsl(w):
            return r_hbm.at[0, pl.ds(c * E2 + (s * WSUB + w) * W, W)]

        def csl(w):
            return c_hbm.at[0, pl.ds((s * WSUB + w) * W, W)]

        @pl.when(s < 15)
        def _():
            pltpu.sync_copy(z_hbm, acc.at[pl.ds(s * 320, 320)])

        @pl.when(s == 15)
        def _():
            pltpu.sync_copy(z_hbm.at[pl.ds(0, 208)],
                            acc.at[pl.ds(4800, 208)])

        plsc.subcore_barrier()

        @pl.loop(0, WSUB)
        def _(w):
            pltpu.sync_copy(rsl(w), idxr0)
            pltpu.sync_copy(csl(w), idxc0)
            pltpu.sync_copy(t_hbm.at[idxc0], buf)
            pltpu.sync_copy(buf, acc.at[idxr0], add=True)

        plsc.subcore_barrier()

        @pl.when(s < 15)
        def _():
            pltpu.sync_copy(acc.at[pl.ds(s * 320, 320)],
                            out_hbm.at[c, pl.ds(s * 320, 320)])

        @pl.when(s == 15)
        def _():
            pltpu.sync_copy(acc.at[pl.ds(4800, 200)],
                            out_hbm.at[c, pl.ds(4800, 200)])

    return k(s128, scr, scc, zeros)


def _compute_final(s128, adj2, ss, c, proj_W, pb8):
    return pl.pallas_call(
        _final_body,
        grid=(NBLK,),
        in_specs=[
            pl.BlockSpec((BN, TW), lambda i: (i, 0)),
            pl.BlockSpec((1, BN, TW), lambda i: (i // 5, i % 5, 0)),
            pl.BlockSpec((KP, KP), lambda i: (0, 0)),
            pl.BlockSpec((KP, D), lambda i: (0, 0)),
            pl.BlockSpec((D, D), lambda i: (0, 0)),
            pl.BlockSpec((8, D), lambda i: (0, 0)),
        ],
        out_specs=[
            pl.BlockSpec((KP, KP), lambda i: (0, 0)),
            pl.BlockSpec((KP, D), lambda i: (0, 0)),
            pl.BlockSpec((8, 128), lambda i: (0, 0)),
        ],
        out_shape=[
            jax.ShapeDtypeStruct((KP, KP), jnp.float32),
            jax.ShapeDtypeStruct((KP, D), jnp.float32),
            jax.ShapeDtypeStruct((8, 128), jnp.float32),
        ],
    )(s128, adj2, ss, c, proj_W, pb8)


def kernel(x, edge_index, assign_W, assign_b, proj_W, proj_b):
    wa_pad = jnp.pad(assign_W, ((0, 0), (0, TW - K)))
    ba_pad = jnp.pad(assign_b, (0, TW - K), constant_values=NEG)
    ba8 = jnp.broadcast_to(ba_pad[None, :], (8, TW))
    pb8 = jnp.broadcast_to(proj_b[None, :], (8, D))
    row = jnp.concatenate([edge_index[0],
                           jnp.full((E2 - E,), N, jnp.int32)])
    col = jnp.concatenate([edge_index[1],
                           jnp.zeros((E2 - E,), jnp.int32)])
    r0 = jnp.where(row < NH, row, NH)
    r1 = row - NH
    r1 = jnp.where((r1 < 0) | (r1 >= NH), NH, r1)
    scr = jnp.concatenate([r0, r1]).reshape(1, 2 * E2)
    scc = col.reshape(1, E2)
    zeros = jnp.zeros((320, TW), jnp.float32)

    s128 = _compute_s(x, wa_pad, ba8)
    adj2 = _sc_segsum(s128, scr, scc, zeros)
    c, ss = _compute_stats(s128, x)
    adj, z, scal = _compute_final(s128, adj2, ss, c, proj_W, pb8)

    S = s128[:, :K]
    Z = z[:K, :]
    adj_new = adj[None, :K, :K]
    mincut_loss = scal[0, 0]
    ortho_loss = scal[0, 1]
    return (Z, adj_new, mincut_loss, ortho_loss, S)


# speedup vs baseline: 1.0388x; 1.0388x over previous
import functools

import jax
import jax.numpy as jnp
from jax import lax
from jax.experimental import pallas as pl
from jax.experimental.pallas import tpu as pltpu
from jax.experimental.pallas import tpu_sc as plsc

N = 10000
E = 160000
D = 256
K = 30
KP = 32
TW = 128
BN = 1000
NBLK = N // BN
W = 640
E2 = 163840
NWIN = E2 // W
WSUB = NWIN // 16
NEG = -1e30


def _s_body(x_ref, wp_ref, b_ref, s_ref):
    logits = jnp.dot(x_ref[...], wp_ref[...],
                     preferred_element_type=jnp.float32) + b_ref[0:1, :]
    m = jnp.max(logits, axis=-1, keepdims=True)
    e = jnp.exp(logits - m)
    s_ref[...] = e / jnp.sum(e, axis=-1, keepdims=True)


def _stats_body(s_ref, x_ref, c_ref, ss_ref):
    i = pl.program_id(0)
    s = s_ref[...][:, :KP]
    ci = lax.broadcasted_iota(jnp.int32, (BN, KP), 1)
    s_aug = jnp.where(ci == KP - 1, 1.0, s)
    c_part = lax.dot_general(s, x_ref[...], (((0,), (0,)), ((), ())),
                             preferred_element_type=jnp.float32)
    ss_part = lax.dot_general(s, s_aug, (((0,), (0,)), ((), ())),
                              preferred_element_type=jnp.float32)

    @pl.when(i == 0)
    def _():
        c_ref[...] = c_part
        ss_ref[...] = ss_part

    @pl.when(i != 0)
    def _():
        c_ref[...] += c_part
        ss_ref[...] += ss_part


def _final_body(s_ref, a0_ref, ss_ref, c_ref, pw_ref, pb_ref,
                adj_ref, z_ref, scal_ref):
    i = pl.program_id(0)
    s = s_ref[...][:, :KP]
    adj_s = a0_ref[0][:, :KP]
    ci = lax.broadcasted_iota(jnp.int32, (BN, KP), 1)
    rs_s = jnp.sum(s, axis=1, keepdims=True)
    rs_a = jnp.sum(adj_s, axis=1, keepdims=True)
    s_aug = jnp.where(ci == KP - 1, rs_s, s)
    a_aug = jnp.where(ci == KP - 1, rs_a, adj_s)
    part = lax.dot_general(s_aug, a_aug, (((0,), (0,)), ((), ())),
                           preferred_element_type=jnp.float32)

    @pl.when(i == 0)
    def _():
        adj_ref[...] = part

    @pl.when(i != 0)
    def _():
        adj_ref[...] += part

    @pl.when(i == NBLK - 1)
    def _():
        r32 = lax.broadcasted_iota(jnp.int32, (KP, KP), 0)
        c32 = lax.broadcasted_iota(jnp.int32, (KP, KP), 1)
        adjn = adj_ref[...]
        cut = jnp.sum(jnp.where((r32 == c32) & (r32 < K), adjn, 0.0))
        vol = jnp.sum(jnp.where((r32 == KP - 1) & (c32 == KP - 1), adjn, 0.0))
        mincut = -cut / (vol + 1e-9)
        ss = ss_ref[...]
        eye = jnp.where((r32 == c32) & (r32 < K), 1.0, 0.0)
        diff = jnp.where((r32 < K) & (c32 < K), ss - eye, 0.0)
        ortho = jnp.sqrt(jnp.sum(diff * diff))
        colsum = ss[:, KP - 1:KP]
        z = jnp.dot(c_ref[...], pw_ref[...],
                    preferred_element_type=jnp.float32)
        z_ref[...] = z + colsum * pb_ref[0:1, :]
        r8 = lax.broadcasted_iota(jnp.int32, (8, 128), 0)
        c8 = lax.broadcasted_iota(jnp.int32, (8, 128), 1)
        scal_ref[...] = jnp.where((r8 == 0) & (c8 == 0), mincut,
                                  jnp.where((r8 == 0) & (c8 == 1), ortho, 0.0))


def _compute_s(x, wa_pad, ba8):
    return pl.pallas_call(
        _s_body,
        grid=(NBLK,),
        in_specs=[
            pl.BlockSpec((BN, D), lambda i: (i, 0)),
            pl.BlockSpec((D, TW), lambda i: (0, 0)),
            pl.BlockSpec((8, TW), lambda i: (0, 0)),
        ],
        out_specs=pl.BlockSpec((BN, TW), lambda i: (i, 0)),
        out_shape=jax.ShapeDtypeStruct((N, TW), jnp.float32),
    )(x, wa_pad, ba8)


def _compute_stats(s128, x):
    return pl.pallas_call(
        _stats_body,
        grid=(NBLK,),
        in_specs=[
            pl.BlockSpec((BN, TW), lambda i: (i, 0)),
            pl.BlockSpec((BN, D), lambda i: (i, 0)),
        ],
        out_specs=[
            pl.BlockSpec((KP, D), lambda i: (0, 0)),
            pl.BlockSpec((KP, KP), lambda i: (0, 0)),
        ],
        out_shape=[
            jax.ShapeDtypeStruct((KP, D), jnp.float32),
            jax.ShapeDtypeStruct((KP, KP), jnp.float32),
        ],
    )(s128, x)


NH = N // 2
ACC = NH + 8


def _sc_segsum(s128, scr, scc, zeros):
    mesh = plsc.VectorSubcoreMesh(core_axis_name="c", subcore_axis_name="s")

    @functools.partial(
        pl.kernel,
        out_type=jax.ShapeDtypeStruct((2, NH, TW), jnp.float32),
        mesh=mesh,
        scratch_types=[
            pltpu.VMEM((W,), jnp.int32),
            pltpu.VMEM((W,), jnp.int32),
            pltpu.VMEM((W,), jnp.int32),
            pltpu.VMEM((W,), jnp.int32),
            pltpu.VMEM((W, TW), jnp.float32),
            pltpu.VMEM_SHARED((ACC, TW), jnp.float32),
            pltpu.SemaphoreType.DMA,
            pltpu.SemaphoreType.DMA,
            pltpu.SemaphoreType.DMA,
            pltpu.SemaphoreType.DMA,
        ],
    )
    def k(t_hbm, r_hbm, c_hbm, z_hbm, out_hbm,
          idxr0, idxr1, idxc0, idxc1, buf, acc, ir0, ir1, ic0, ic1):
        idxr = (idxr0, idxr1)
        idxc = (idxc0, idxc1)
        ir = (ir0, ir1)
        ic = (ic0, ic1)
        c = lax.axis_index("c")
        s = lax.axis_index("s")

        def rsl(w):
            return r_hbm.at[0, pl.ds(c * E2 + (s * WSUB + w) * W, W)]

        def csl(w):
            return c_hbm.at[0, pl.ds((s * WSUB + w) * W, W)]

        @pl.when(s < 15)
        def _():
            pltpu.sync_copy(z_hbm, acc.at[pl.ds(s * 320, 320)])

        @pl.when(s == 15)
        def _():
            pltpu.sync_copy(z_hbm.at[pl.ds(0, 208)],
                            acc.at[pl.ds(4800, 208)])

        plsc.subcore_barrier()

        @pl.loop(0, WSUB)
        def _(w):
            pltpu.sync_copy(rsl(w), idxr0)
            pltpu.sync_copy(csl(w), idxc0)
            pltpu.sync_copy(t_hbm.at[idxc0], buf)
            pltpu.sync_copy(buf, acc.at[idxr0], add=True)

        plsc.subcore_barrier()

        @pl.when(s < 15)
        def _():
            pltpu.sync_copy(acc.at[pl.ds(s * 320, 320)],
                            out_hbm.at[c, pl.ds(s * 320, 320)])

        @pl.when(s == 15)
        def _():
            pltpu.sync_copy(acc.at[pl.ds(4800, 200)],
                            out_hbm.at[c, pl.ds(4800, 200)])

    return k(s128, scr, scc, zeros)


def _compute_final(s128, adj2, ss, c, proj_W, pb8):
    return pl.pallas_call(
        _final_body,
        grid=(NBLK,),
        in_specs=[
            pl.BlockSpec((BN, TW), lambda i: (i, 0)),
            pl.BlockSpec((1, BN, TW), lambda i: (i // 5, i % 5, 0)),
            pl.BlockSpec((KP, KP), lambda i: (0, 0)),
            pl.BlockSpec((KP, D), lambda i: (0, 0)),
            pl.BlockSpec((D, D), lambda i: (0, 0)),
            pl.BlockSpec((8, D), lambda i: (0, 0)),
        ],
        out_specs=[
            pl.BlockSpec((KP, KP), lambda i: (0, 0)),
            pl.BlockSpec((KP, D), lambda i: (0, 0)),
            pl.BlockSpec((8, 128), lambda i: (0, 0)),
        ],
        out_shape=[
            jax.ShapeDtypeStruct((KP, KP), jnp.float32),
            jax.ShapeDtypeStruct((KP, D), jnp.float32),
            jax.ShapeDtypeStruct((8, 128), jnp.float32),
        ],
    )(s128, adj2, ss, c, proj_W, pb8)


def kernel(x, edge_index, assign_W, assign_b, proj_W, proj_b):
    wa_pad = jnp.pad(assign_W, ((0, 0), (0, TW - K)))
    ba_pad = jnp.pad(assign_b, (0, TW - K), constant_values=NEG)
    ba8 = jnp.broadcast_to(ba_pad[None, :], (8, TW))
    pb8 = jnp.broadcast_to(proj_b[None, :], (8, D))
    row = jnp.concatenate([edge_index[0],
                           jnp.full((E2 - E,), N, jnp.int32)])
    col = jnp.concatenate([edge_index[1],
                           jnp.zeros((E2 - E,), jnp.int32)])
    dump = NH + (jnp.arange(E2, dtype=jnp.int32) & 7)
    r0 = jnp.where(row < NH, row, dump)
    r1 = row - NH
    r1 = jnp.where((r1 < 0) | (r1 >= NH), dump, r1)
    scr = jnp.concatenate([r0, r1]).reshape(1, 2 * E2)
    scc = col.reshape(1, E2)
    zeros = jnp.zeros((320, TW), jnp.float32)

    s128 = _compute_s(x, wa_pad, ba8)
    adj2 = _sc_segsum(s128, scr, scc, zeros)
    c, ss = _compute_stats(s128, x)
    adj, z, scal = _compute_final(s128, adj2, ss, c, proj_W, pb8)

    S = s128[:, :K]
    Z = z[:K, :]
    adj_new = adj[None, :K, :K]
    mincut_loss = scal[0, 0]
    ortho_loss = scal[0, 1]
    return (Z, adj_new, mincut_loss, ortho_loss, S)


# speedup vs baseline: 2.2718x; 2.1871x over previous
import functools

import jax
import jax.numpy as jnp
from jax import lax
from jax.experimental import pallas as pl
from jax.experimental.pallas import tpu as pltpu
from jax.experimental.pallas import tpu_sc as plsc

N = 10000
E = 160000
D = 256
K = 30
KP = 32
TW = 128
BN = 1000
NBLK = N // BN
W = 640
NWIN = E // W
NEG = -1e30


def _s_body(x_ref, wp_ref, b_ref, s_ref):
    logits = jnp.dot(x_ref[...], wp_ref[...],
                     preferred_element_type=jnp.float32) + b_ref[0:1, :]
    m = jnp.max(logits, axis=-1, keepdims=True)
    e = jnp.exp(logits - m)
    s_ref[...] = e / jnp.sum(e, axis=-1, keepdims=True)


def _stats_body(s_ref, x_ref, c_ref, ss_ref):
    i = pl.program_id(0)
    s = s_ref[...][:, :KP]
    ci = lax.broadcasted_iota(jnp.int32, (BN, KP), 1)
    s_aug = jnp.where(ci == KP - 1, 1.0, s)
    c_part = lax.dot_general(s, x_ref[...], (((0,), (0,)), ((), ())),
                             preferred_element_type=jnp.float32)
    ss_part = lax.dot_general(s, s_aug, (((0,), (0,)), ((), ())),
                              preferred_element_type=jnp.float32)

    @pl.when(i == 0)
    def _():
        c_ref[...] = c_part
        ss_ref[...] = ss_part

    @pl.when(i != 0)
    def _():
        c_ref[...] += c_part
        ss_ref[...] += ss_part


def _final_body(s_ref, a0_ref, ss_ref, c_ref, pw_ref, pb_ref,
                adj_ref, z_ref, scal_ref):
    i = pl.program_id(0)
    s = s_ref[...][:, :KP]
    adj_s = a0_ref[0][:, :KP]
    ci = lax.broadcasted_iota(jnp.int32, (BN, KP), 1)
    rs_s = jnp.sum(s, axis=1, keepdims=True)
    rs_a = jnp.sum(adj_s, axis=1, keepdims=True)
    s_aug = jnp.where(ci == KP - 1, rs_s, s)
    a_aug = jnp.where(ci == KP - 1, rs_a, adj_s)
    part = lax.dot_general(s_aug, a_aug, (((0,), (0,)), ((), ())),
                           preferred_element_type=jnp.float32)

    @pl.when(i == 0)
    def _():
        adj_ref[...] = part

    @pl.when(i != 0)
    def _():
        adj_ref[...] += part

    @pl.when(i == NBLK - 1)
    def _():
        r32 = lax.broadcasted_iota(jnp.int32, (KP, KP), 0)
        c32 = lax.broadcasted_iota(jnp.int32, (KP, KP), 1)
        adjn = adj_ref[...]
        cut = jnp.sum(jnp.where((r32 == c32) & (r32 < K), adjn, 0.0))
        vol = jnp.sum(jnp.where((r32 == KP - 1) & (c32 == KP - 1), adjn, 0.0))
        mincut = -cut / (vol + 1e-9)
        ss = ss_ref[...]
        eye = jnp.where((r32 == c32) & (r32 < K), 1.0, 0.0)
        diff = jnp.where((r32 < K) & (c32 < K), ss - eye, 0.0)
        ortho = jnp.sqrt(jnp.sum(diff * diff))
        colsum = ss[:, KP - 1:KP]
        z = jnp.dot(c_ref[...], pw_ref[...],
                    preferred_element_type=jnp.float32)
        z_ref[...] = z + colsum * pb_ref[0:1, :]
        r8 = lax.broadcasted_iota(jnp.int32, (8, 128), 0)
        c8 = lax.broadcasted_iota(jnp.int32, (8, 128), 1)
        scal_ref[...] = jnp.where((r8 == 0) & (c8 == 0), mincut,
                                  jnp.where((r8 == 0) & (c8 == 1), ortho, 0.0))


def _compute_s(x, wa_pad, ba8):
    return pl.pallas_call(
        _s_body,
        grid=(NBLK,),
        in_specs=[
            pl.BlockSpec((BN, D), lambda i: (i, 0)),
            pl.BlockSpec((D, TW), lambda i: (0, 0)),
            pl.BlockSpec((8, TW), lambda i: (0, 0)),
        ],
        out_specs=pl.BlockSpec((BN, TW), lambda i: (i, 0)),
        out_shape=jax.ShapeDtypeStruct((N, TW), jnp.float32),
    )(x, wa_pad, ba8)


def _compute_stats(s128, x):
    return pl.pallas_call(
        _stats_body,
        grid=(NBLK,),
        in_specs=[
            pl.BlockSpec((BN, TW), lambda i: (i, 0)),
            pl.BlockSpec((BN, D), lambda i: (i, 0)),
        ],
        out_specs=[
            pl.BlockSpec((KP, D), lambda i: (0, 0)),
            pl.BlockSpec((KP, KP), lambda i: (0, 0)),
        ],
        out_shape=[
            jax.ShapeDtypeStruct((KP, D), jnp.float32),
            jax.ShapeDtypeStruct((KP, KP), jnp.float32),
        ],
    )(s128, x)


NH = N // 2
ACC = NH + 8


def _sc_segsum(s128, idx2, zeros):
    mesh = plsc.VectorSubcoreMesh(core_axis_name="c", subcore_axis_name="s")

    @functools.partial(
        pl.kernel,
        out_type=jax.ShapeDtypeStruct((2, NH, TW), jnp.float32),
        mesh=mesh,
        scratch_types=[
            pltpu.VMEM((W,), jnp.int32),
            pltpu.VMEM((W,), jnp.int32),
            pltpu.VMEM((W, TW), jnp.float32),
            pltpu.VMEM_SHARED((ACC, TW), jnp.float32),
        ],
    )
    def k(t_hbm, i_hbm, z_hbm, out_hbm, idxr_v, idxc_v, buf_v, acc):
        c = lax.axis_index("c")
        s = lax.axis_index("s")

        @pl.when(s < 15)
        def _():
            pltpu.sync_copy(z_hbm, acc.at[pl.ds(s * 320, 320)])

        @pl.when(s == 15)
        def _():
            pltpu.sync_copy(z_hbm.at[pl.ds(0, 208)],
                            acc.at[pl.ds(4800, 208)])

        plsc.subcore_barrier()

        @pl.loop(0, (NWIN + 15) // 16)
        def _(w):
            g = w * 16 + s

            @pl.when(g < NWIN)
            def _():
                base = g * W
                pltpu.sync_copy(i_hbm.at[0, pl.ds(c * E + base, W)], idxr_v)
                pltpu.sync_copy(i_hbm.at[0, pl.ds(2 * E + base, W)], idxc_v)
                pltpu.sync_copy(t_hbm.at[idxc_v], buf_v)
                pltpu.sync_copy(buf_v, acc.at[idxr_v], add=True)

        plsc.subcore_barrier()

        @pl.when(s < 15)
        def _():
            pltpu.sync_copy(acc.at[pl.ds(s * 320, 320)],
                            out_hbm.at[c, pl.ds(s * 320, 320)])

        @pl.when(s == 15)
        def _():
            pltpu.sync_copy(acc.at[pl.ds(4800, 200)],
                            out_hbm.at[c, pl.ds(4800, 200)])

    return k(s128, idx2, zeros)


def _compute_final(s128, adj2, ss, c, proj_W, pb8):
    return pl.pallas_call(
        _final_body,
        grid=(NBLK,),
        in_specs=[
            pl.BlockSpec((BN, TW), lambda i: (i, 0)),
            pl.BlockSpec((1, BN, TW), lambda i: (i // 5, i % 5, 0)),
            pl.BlockSpec((KP, KP), lambda i: (0, 0)),
            pl.BlockSpec((KP, D), lambda i: (0, 0)),
            pl.BlockSpec((D, D), lambda i: (0, 0)),
            pl.BlockSpec((8, D), lambda i: (0, 0)),
        ],
        out_specs=[
            pl.BlockSpec((KP, KP), lambda i: (0, 0)),
            pl.BlockSpec((KP, D), lambda i: (0, 0)),
            pl.BlockSpec((8, 128), lambda i: (0, 0)),
        ],
        out_shape=[
            jax.ShapeDtypeStruct((KP, KP), jnp.float32),
            jax.ShapeDtypeStruct((KP, D), jnp.float32),
            jax.ShapeDtypeStruct((8, 128), jnp.float32),
        ],
    )(s128, adj2, ss, c, proj_W, pb8)


def kernel(x, edge_index, assign_W, assign_b, proj_W, proj_b):
    wa_pad = jnp.pad(assign_W, ((0, 0), (0, TW - K)))
    ba_pad = jnp.pad(assign_b, (0, TW - K), constant_values=NEG)
    ba8 = jnp.broadcast_to(ba_pad[None, :], (8, TW))
    pb8 = jnp.broadcast_to(proj_b[None, :], (8, D))
    row = edge_index[0]
    col = edge_index[1]
    dump = NH + (jnp.arange(E, dtype=jnp.int32) & 7)
    r0 = jnp.where(row < NH, row, dump)
    r1 = row - NH
    r1 = jnp.where((r1 < 0) | (r1 >= NH), dump, r1)
    idx2 = jnp.concatenate([r0, r1, col]).reshape(1, 3 * E)
    zeros = jnp.zeros((320, TW), jnp.float32)

    s128 = _compute_s(x, wa_pad, ba8)
    adj2 = _sc_segsum(s128, idx2, zeros)
    c, ss = _compute_stats(s128, x)
    adj, z, scal = _compute_final(s128, adj2, ss, c, proj_W, pb8)

    S = s128[:, :K]
    Z = z[:K, :]
    adj_new = adj[None, :K, :K]
    mincut_loss = scal[0, 0]
    ortho_loss = scal[0, 1]
    return (Z, adj_new, mincut_loss, ortho_loss, S)


# speedup vs baseline: 2.3388x; 1.0295x over previous
import functools

import jax
import jax.numpy as jnp
from jax import lax
from jax.experimental import pallas as pl
from jax.experimental.pallas import tpu as pltpu
from jax.experimental.pallas import tpu_sc as plsc

N = 10000
E = 160000
D = 256
K = 30
KP = 32
TW = 128
BN = 1000
NBLK = N // BN
W = 640
NWIN = E // W
NEG = -1e30


def _s_body(x_ref, wp_ref, b_ref, s_ref):
    logits = jnp.dot(x_ref[...], wp_ref[...],
                     preferred_element_type=jnp.float32) + b_ref[0:1, :]
    m = jnp.max(logits, axis=-1, keepdims=True)
    e = jnp.exp(logits - m)
    s_ref[...] = e / jnp.sum(e, axis=-1, keepdims=True)


def _stats_body(s_ref, x_ref, c_ref, ss_ref):
    i = pl.program_id(0)
    s = s_ref[...][:, :KP]
    ci = lax.broadcasted_iota(jnp.int32, (BN, KP), 1)
    s_aug = jnp.where(ci == KP - 1, 1.0, s)
    c_part = lax.dot_general(s, x_ref[...], (((0,), (0,)), ((), ())),
                             preferred_element_type=jnp.float32)
    ss_part = lax.dot_general(s, s_aug, (((0,), (0,)), ((), ())),
                              preferred_element_type=jnp.float32)

    @pl.when(i == 0)
    def _():
        c_ref[...] = c_part
        ss_ref[...] = ss_part

    @pl.when(i != 0)
    def _():
        c_ref[...] += c_part
        ss_ref[...] += ss_part


def _final_body(s_ref, a0_ref, ss_ref, c_ref, pw_ref, pb_ref,
                adj_ref, z_ref, scal_ref):
    i = pl.program_id(0)
    s = s_ref[...][:, :KP]
    adj_s = a0_ref[0][:, :KP]
    ci = lax.broadcasted_iota(jnp.int32, (BN, KP), 1)
    rs_s = jnp.sum(s, axis=1, keepdims=True)
    rs_a = jnp.sum(adj_s, axis=1, keepdims=True)
    s_aug = jnp.where(ci == KP - 1, rs_s, s)
    a_aug = jnp.where(ci == KP - 1, rs_a, adj_s)
    part = lax.dot_general(s_aug, a_aug, (((0,), (0,)), ((), ())),
                           preferred_element_type=jnp.float32)

    @pl.when(i == 0)
    def _():
        adj_ref[...] = part

    @pl.when(i != 0)
    def _():
        adj_ref[...] += part

    @pl.when(i == NBLK - 1)
    def _():
        r32 = lax.broadcasted_iota(jnp.int32, (KP, KP), 0)
        c32 = lax.broadcasted_iota(jnp.int32, (KP, KP), 1)
        adjn = adj_ref[...]
        cut = jnp.sum(jnp.where((r32 == c32) & (r32 < K), adjn, 0.0))
        vol = jnp.sum(jnp.where((r32 == KP - 1) & (c32 == KP - 1), adjn, 0.0))
        mincut = -cut / (vol + 1e-9)
        ss = ss_ref[...]
        eye = jnp.where((r32 == c32) & (r32 < K), 1.0, 0.0)
        diff = jnp.where((r32 < K) & (c32 < K), ss - eye, 0.0)
        ortho = jnp.sqrt(jnp.sum(diff * diff))
        colsum = ss[:, KP - 1:KP]
        z = jnp.dot(c_ref[...], pw_ref[...],
                    preferred_element_type=jnp.float32)
        z_ref[...] = z + colsum * pb_ref[0:1, :]
        r8 = lax.broadcasted_iota(jnp.int32, (8, 128), 0)
        c8 = lax.broadcasted_iota(jnp.int32, (8, 128), 1)
        scal_ref[...] = jnp.where((r8 == 0) & (c8 == 0), mincut,
                                  jnp.where((r8 == 0) & (c8 == 1), ortho, 0.0))


def _compute_s(x, wa_pad, ba8):
    return pl.pallas_call(
        _s_body,
        grid=(NBLK,),
        in_specs=[
            pl.BlockSpec((BN, D), lambda i: (i, 0)),
            pl.BlockSpec((D, TW), lambda i: (0, 0)),
            pl.BlockSpec((8, TW), lambda i: (0, 0)),
        ],
        out_specs=pl.BlockSpec((BN, TW), lambda i: (i, 0)),
        out_shape=jax.ShapeDtypeStruct((N, TW), jnp.float32),
    )(x, wa_pad, ba8)


def _compute_stats(s128, x):
    return pl.pallas_call(
        _stats_body,
        grid=(NBLK,),
        in_specs=[
            pl.BlockSpec((BN, TW), lambda i: (i, 0)),
            pl.BlockSpec((BN, D), lambda i: (i, 0)),
        ],
        out_specs=[
            pl.BlockSpec((KP, D), lambda i: (0, 0)),
            pl.BlockSpec((KP, KP), lambda i: (0, 0)),
        ],
        out_shape=[
            jax.ShapeDtypeStruct((KP, D), jnp.float32),
            jax.ShapeDtypeStruct((KP, KP), jnp.float32),
        ],
    )(s128, x)


NH = N // 2
ACC = NH + 8


def _sc_segsum(s128, idx2, zeros):
    mesh = plsc.VectorSubcoreMesh(core_axis_name="c", subcore_axis_name="s")

    @functools.partial(
        pl.kernel,
        out_type=jax.ShapeDtypeStruct((2, NH, TW), jnp.float32),
        mesh=mesh,
        scratch_types=[
            pltpu.VMEM((W,), jnp.int32),
            pltpu.VMEM((W,), jnp.int32),
            pltpu.VMEM((W, TW), jnp.float32),
            pltpu.VMEM_SHARED((ACC, TW), jnp.float32),
        ],
    )
    def k(t_hbm, i_hbm, z_hbm, out_hbm, idxr_v, idxc_v, buf_v, acc):
        c = lax.axis_index("c")
        s = lax.axis_index("s")

        @pl.when(s < 15)
        def _():
            pltpu.sync_copy(z_hbm, acc.at[pl.ds(s * 320, 320)])

        @pl.when(s == 15)
        def _():
            pltpu.sync_copy(z_hbm.at[pl.ds(0, 208)],
                            acc.at[pl.ds(4800, 208)])

        plsc.subcore_barrier()

        @pl.loop(0, (NWIN + 15) // 16)
        def _(w):
            g = w * 16 + s

            @pl.when(g < NWIN)
            def _():
                base = g * W
                pltpu.sync_copy(i_hbm.at[pl.ds(c * E + base, W)], idxr_v)
                pltpu.sync_copy(i_hbm.at[pl.ds(2 * E + base, W)], idxc_v)
                pltpu.sync_copy(t_hbm.at[idxc_v], buf_v)
                pltpu.sync_copy(buf_v, acc.at[idxr_v], add=True)

        plsc.subcore_barrier()

        @pl.when(s < 15)
        def _():
            pltpu.sync_copy(acc.at[pl.ds(s * 320, 320)],
                            out_hbm.at[c, pl.ds(s * 320, 320)])

        @pl.when(s == 15)
        def _():
            pltpu.sync_copy(acc.at[pl.ds(4800, 200)],
                            out_hbm.at[c, pl.ds(4800, 200)])

    return k(s128, idx2, zeros)


def _compute_final(s128, adj2, ss, c, proj_W, pb8):
    return pl.pallas_call(
        _final_body,
        grid=(NBLK,),
        in_specs=[
            pl.BlockSpec((BN, TW), lambda i: (i, 0)),
            pl.BlockSpec((1, BN, TW), lambda i: (i // 5, i % 5, 0)),
            pl.BlockSpec((KP, KP), lambda i: (0, 0)),
            pl.BlockSpec((KP, D), lambda i: (0, 0)),
            pl.BlockSpec((D, D), lambda i: (0, 0)),
            pl.BlockSpec((8, D), lambda i: (0, 0)),
        ],
        out_specs=[
            pl.BlockSpec((KP, KP), lambda i: (0, 0)),
            pl.BlockSpec((KP, D), lambda i: (0, 0)),
            pl.BlockSpec((8, 128), lambda i: (0, 0)),
        ],
        out_shape=[
            jax.ShapeDtypeStruct((KP, KP), jnp.float32),
            jax.ShapeDtypeStruct((KP, D), jnp.float32),
            jax.ShapeDtypeStruct((8, 128), jnp.float32),
        ],
    )(s128, adj2, ss, c, proj_W, pb8)


def kernel(x, edge_index, assign_W, assign_b, proj_W, proj_b):
    wa_pad = jnp.pad(assign_W, ((0, 0), (0, TW - K)))
    ba_pad = jnp.pad(assign_b, (0, TW - K), constant_values=NEG)
    ba8 = jnp.broadcast_to(ba_pad[None, :], (8, TW))
    pb8 = jnp.broadcast_to(proj_b[None, :], (8, D))
    ei = edge_index.reshape(2 * E)
    row = ei[:E]
    col = ei[E:]
    dump = NH + (jnp.arange(E, dtype=jnp.int32) & 7)
    r0 = jnp.where(row < NH, row, dump)
    r1 = row - NH
    r1 = jnp.where((r1 < 0) | (r1 >= NH), dump, r1)
    idx2 = jnp.concatenate([r0, r1, col])
    zeros = jnp.zeros((320, TW), jnp.float32)

    s128 = _compute_s(x, wa_pad, ba8)
    adj2 = _sc_segsum(s128, idx2, zeros)
    c, ss = _compute_stats(s128, x)
    adj, z, scal = _compute_final(s128, adj2, ss, c, proj_W, pb8)

    S = s128[:, :K]
    Z = z[:K, :]
    adj_new = adj[None, :K, :K]
    mincut_loss = scal[0, 0]
    ortho_loss = scal[0, 1]
    return (Z, adj_new, mincut_loss, ortho_loss, S)


# speedup vs baseline: 2.5473x; 1.0892x over previous
import functools

import jax
import jax.numpy as jnp
from jax import lax
from jax.experimental import pallas as pl
from jax.experimental.pallas import tpu as pltpu
from jax.experimental.pallas import tpu_sc as plsc

N = 10000
E = 160000
D = 256
K = 30
KP = 32
TW = 128
BN = 1000
NBLK = N // BN
W = 640
NWIN = E // W
NEG = -1e30


def _s_body(x_ref, wp_ref, b_ref, s_ref):
    logits = jnp.dot(x_ref[...], wp_ref[...],
                     preferred_element_type=jnp.float32) + b_ref[0:1, :]
    m = jnp.max(logits, axis=-1, keepdims=True)
    e = jnp.exp(logits - m)
    s_ref[...] = e / jnp.sum(e, axis=-1, keepdims=True)


def _stats_body(s_ref, x_ref, c_ref, ss_ref):
    i = pl.program_id(0)
    s = s_ref[...][:, :KP]
    ci = lax.broadcasted_iota(jnp.int32, (BN, KP), 1)
    s_aug = jnp.where(ci == KP - 1, 1.0, s)
    c_part = lax.dot_general(s, x_ref[...], (((0,), (0,)), ((), ())),
                             preferred_element_type=jnp.float32)
    ss_part = lax.dot_general(s, s_aug, (((0,), (0,)), ((), ())),
                              preferred_element_type=jnp.float32)

    @pl.when(i == 0)
    def _():
        c_ref[...] = c_part
        ss_ref[...] = ss_part

    @pl.when(i != 0)
    def _():
        c_ref[...] += c_part
        ss_ref[...] += ss_part


def _final_body(s_ref, a0_ref, ss_ref, c_ref, pw_ref, pb_ref,
                adj_ref, z_ref, scal_ref):
    i = pl.program_id(0)
    s = s_ref[...][:, :KP]
    adj_s = a0_ref[0][:, :KP]
    ci = lax.broadcasted_iota(jnp.int32, (BN, KP), 1)
    rs_s = jnp.sum(s, axis=1, keepdims=True)
    rs_a = jnp.sum(adj_s, axis=1, keepdims=True)
    s_aug = jnp.where(ci == KP - 1, rs_s, s)
    a_aug = jnp.where(ci == KP - 1, rs_a, adj_s)
    part = lax.dot_general(s_aug, a_aug, (((0,), (0,)), ((), ())),
                           preferred_element_type=jnp.float32)

    @pl.when(i == 0)
    def _():
        adj_ref[...] = part

    @pl.when(i != 0)
    def _():
        adj_ref[...] += part

    @pl.when(i == NBLK - 1)
    def _():
        r32 = lax.broadcasted_iota(jnp.int32, (KP, KP), 0)
        c32 = lax.broadcasted_iota(jnp.int32, (KP, KP), 1)
        adjn = adj_ref[...]
        cut = jnp.sum(jnp.where((r32 == c32) & (r32 < K), adjn, 0.0))
        vol = jnp.sum(jnp.where((r32 == KP - 1) & (c32 == KP - 1), adjn, 0.0))
        mincut = -cut / (vol + 1e-9)
        ss = ss_ref[...]
        eye = jnp.where((r32 == c32) & (r32 < K), 1.0, 0.0)
        diff = jnp.where((r32 < K) & (c32 < K), ss - eye, 0.0)
        ortho = jnp.sqrt(jnp.sum(diff * diff))
        colsum = ss[:, KP - 1:KP]
        z = jnp.dot(c_ref[...], pw_ref[...],
                    preferred_element_type=jnp.float32)
        z_ref[...] = z + colsum * pb_ref[0:1, :]
        r8 = lax.broadcasted_iota(jnp.int32, (8, 128), 0)
        c8 = lax.broadcasted_iota(jnp.int32, (8, 128), 1)
        scal_ref[...] = jnp.where((r8 == 0) & (c8 == 0), mincut,
                                  jnp.where((r8 == 0) & (c8 == 1), ortho, 0.0))


def _compute_s(x, wa_pad, ba8):
    return pl.pallas_call(
        _s_body,
        grid=(NBLK,),
        in_specs=[
            pl.BlockSpec((BN, D), lambda i: (i, 0)),
            pl.BlockSpec((D, TW), lambda i: (0, 0)),
            pl.BlockSpec((8, TW), lambda i: (0, 0)),
        ],
        out_specs=pl.BlockSpec((BN, TW), lambda i: (i, 0)),
        out_shape=jax.ShapeDtypeStruct((N, TW), jnp.float32),
    )(x, wa_pad, ba8)


def _compute_stats(s128, x):
    return pl.pallas_call(
        _stats_body,
        grid=(NBLK,),
        in_specs=[
            pl.BlockSpec((BN, TW), lambda i: (i, 0)),
            pl.BlockSpec((BN, D), lambda i: (i, 0)),
        ],
        out_specs=[
            pl.BlockSpec((KP, D), lambda i: (0, 0)),
            pl.BlockSpec((KP, KP), lambda i: (0, 0)),
        ],
        out_shape=[
            jax.ShapeDtypeStruct((KP, D), jnp.float32),
            jax.ShapeDtypeStruct((KP, KP), jnp.float32),
        ],
    )(s128, x)


NH = N // 2
ACC = NH + 8


def _sc_segsum(s128, idx2, zeros):
    mesh = plsc.VectorSubcoreMesh(core_axis_name="c", subcore_axis_name="s")

    @functools.partial(
        pl.kernel,
        out_type=jax.ShapeDtypeStruct((2, NH, TW), jnp.float32),
        mesh=mesh,
        scratch_types=[
            pltpu.VMEM((W,), jnp.int32),
            pltpu.VMEM((W,), jnp.int32),
            pltpu.VMEM((W,), jnp.int32),
            pltpu.VMEM((W,), jnp.int32),
            pltpu.VMEM((W, TW), jnp.float32),
            pltpu.VMEM_SHARED((ACC, TW), jnp.float32),
            pltpu.SemaphoreType.DMA,
            pltpu.SemaphoreType.DMA,
            pltpu.SemaphoreType.DMA,
            pltpu.SemaphoreType.DMA,
        ],
    )
    def k(t_hbm, i_hbm, z_hbm, out_hbm,
          idxr0, idxr1, idxc0, idxc1, buf_v, acc, ir0, ir1, ic0, ic1):
        idxr = (idxr0, idxr1)
        idxc = (idxc0, idxc1)
        ir = (ir0, ir1)
        ic = (ic0, ic1)
        c = lax.axis_index("c")
        s = lax.axis_index("s")

        def rsl(g):
            return i_hbm.at[pl.ds(c * E + g * W, W)]

        def csl(g):
            return i_hbm.at[pl.ds(2 * E + g * W, W)]

        for u in (0, 1):
            pltpu.async_copy(rsl(u * 16 + s), idxr[u], ir[u])
            pltpu.async_copy(csl(u * 16 + s), idxc[u], ic[u])

        @pl.when(s < 15)
        def _():
            pltpu.sync_copy(z_hbm, acc.at[pl.ds(s * 320, 320)])

        @pl.when(s == 15)
        def _():
            pltpu.sync_copy(z_hbm.at[pl.ds(0, 208)],
                            acc.at[pl.ds(4800, 208)])

        plsc.subcore_barrier()

        @pl.loop(0, 8)
        def _(t):
            for u in (0, 1):
                g = (2 * t + u) * 16 + s

                @pl.when(g < NWIN)
                def _():
                    pltpu.make_async_copy(rsl(g), idxr[u], ir[u]).wait()
                    pltpu.make_async_copy(csl(g), idxc[u], ic[u]).wait()
                    pltpu.sync_copy(t_hbm.at[idxc[u]], buf_v)
                    pltpu.sync_copy(buf_v, acc.at[idxr[u]], add=True)

                    @pl.when(g + 32 < NWIN)
                    def _():
                        pltpu.async_copy(rsl(g + 32), idxr[u], ir[u])
                        pltpu.async_copy(csl(g + 32), idxc[u], ic[u])

        plsc.subcore_barrier()

        @pl.when(s < 15)
        def _():
            pltpu.sync_copy(acc.at[pl.ds(s * 320, 320)],
                            out_hbm.at[c, pl.ds(s * 320, 320)])

        @pl.when(s == 15)
        def _():
            pltpu.sync_copy(acc.at[pl.ds(4800, 200)],
                            out_hbm.at[c, pl.ds(4800, 200)])

    return k(s128, idx2, zeros)


def _compute_final(s128, adj2, ss, c, proj_W, pb8):
    return pl.pallas_call(
        _final_body,
        grid=(NBLK,),
        in_specs=[
            pl.BlockSpec((BN, TW), lambda i: (i, 0)),
            pl.BlockSpec((1, BN, TW), lambda i: (i // 5, i % 5, 0)),
            pl.BlockSpec((KP, KP), lambda i: (0, 0)),
            pl.BlockSpec((KP, D), lambda i: (0, 0)),
            pl.BlockSpec((D, D), lambda i: (0, 0)),
            pl.BlockSpec((8, D), lambda i: (0, 0)),
        ],
        out_specs=[
            pl.BlockSpec((KP, KP), lambda i: (0, 0)),
            pl.BlockSpec((KP, D), lambda i: (0, 0)),
            pl.BlockSpec((8, 128), lambda i: (0, 0)),
        ],
        out_shape=[
            jax.ShapeDtypeStruct((KP, KP), jnp.float32),
            jax.ShapeDtypeStruct((KP, D), jnp.float32),
            jax.ShapeDtypeStruct((8, 128), jnp.float32),
        ],
    )(s128, adj2, ss, c, proj_W, pb8)


def kernel(x, edge_index, assign_W, assign_b, proj_W, proj_b):
    wa_pad = jnp.pad(assign_W, ((0, 0), (0, TW - K)))
    ba_pad = jnp.pad(assign_b, (0, TW - K), constant_values=NEG)
    ba8 = jnp.broadcast_to(ba_pad[None, :], (8, TW))
    pb8 = jnp.broadcast_to(proj_b[None, :], (8, D))
    ei = edge_index.reshape(2 * E)
    row = ei[:E]
    col = ei[E:]
    dump = NH + (jnp.arange(E, dtype=jnp.int32) & 7)
    r0 = jnp.where(row < NH, row, dump)
    r1 = row - NH
    r1 = jnp.where((r1 < 0) | (r1 >= NH), dump, r1)
    idx2 = jnp.concatenate([r0, r1, col])
    zeros = jnp.zeros((320, TW), jnp.float32)

    s128 = _compute_s(x, wa_pad, ba8)
    adj2 = _sc_segsum(s128, idx2, zeros)
    c, ss = _compute_stats(s128, x)
    adj, z, scal = _compute_final(s128, adj2, ss, c, proj_W, pb8)

    S = s128[:, :K]
    Z = z[:K, :]
    adj_new = adj[None, :K, :K]
    mincut_loss = scal[0, 0]
    ortho_loss = scal[0, 1]
    return (Z, adj_new, mincut_loss, ortho_loss, S)
